# Initial kernel scaffold; baseline (speedup 1.0000x reference)
import jax
import jax.numpy as jnp
from jax.experimental import pallas as pl


def kernel(x, edges_index, edges_weight, bn_g, bn_b, Wg, bg, W1, b1, W2, b2):
    raise NotImplementedError("write your pallas kernel here")


# speedup vs baseline: 7.5794x; 7.5794x over previous
import functools

import jax
import jax.numpy as jnp
from jax import lax
from jax.experimental import pallas as pl
from jax.experimental.pallas import tpu as pltpu
from jax.experimental.pallas import tpu_sc as plsc

N = 10000
E = 320000
K = 3
D = 128
H1 = 128
H2 = 64
N_LAYER = 3

NC = 2
NS = 16
NW = NC * NS
NPAD = 10240
EPW = E // NW
C = 80
NCHUNK = EPW // C
SUB = 25
ZR = 64

def _wid():
    c = lax.axis_index("c")
    s = lax.axis_index("s")
    return s * NC + c, c, s


def _mesh():
    return plsc.VectorSubcoreMesh(
        core_axis_name="c", subcore_axis_name="s",
        num_cores=NC, num_subcores=NS)


@functools.cache
def _sc_deg_kernel():
    return pl.kernel(
        _sc_deg_body,
        out_type=jax.ShapeDtypeStruct((NC, K, NPAD), jnp.float32),
        mesh=_mesh(),
        compiler_params=pltpu.CompilerParams(use_tc_tiling_on_sc=False, needs_layout_passes=False),
        scratch_types=[
            pltpu.VMEM((NCHUNK, C), jnp.int32),
            pltpu.VMEM((NCHUNK, C), jnp.float32),
            pltpu.VMEM((C,), jnp.int32),
            pltpu.VMEM((C,), jnp.float32),
            pltpu.VMEM((640,), jnp.float32),
            pltpu.VMEM_SHARED((NPAD,), jnp.float32),
        ],
    )


def _sc_deg_body(dst_hbm, w_hbm, deg_out, dst_v, w_v, didx_v, w1_v, zb_v,
                 acc_sh):
    w, c, s = _wid()
    zero16 = jnp.zeros((16,), jnp.float32)

    def zb_body(i, _):
        zb_v[pl.ds(i * 16, 16)] = zero16
        return 0
    lax.fori_loop(0, 640 // 16, zb_body, 0)

    for k in range(K):
        pltpu.sync_copy(zb_v, acc_sh.at[pl.ds(s * 640, 640)])
        plsc.subcore_barrier()
        pltpu.sync_copy(dst_hbm.at[k, pl.ds(w * NCHUNK, NCHUNK)], dst_v)
        pltpu.sync_copy(w_hbm.at[k, pl.ds(w * NCHUNK, NCHUNK)], w_v)

        def body(j, _):
            for t in range(C // 16):
                didx_v[pl.ds(t * 16, 16)] = dst_v[j, pl.ds(t * 16, 16)]
                w1_v[pl.ds(t * 16, 16)] = w_v[j, pl.ds(t * 16, 16)]
            pltpu.sync_copy(w1_v, acc_sh.at[didx_v], add=True)
            return 0
        lax.fori_loop(0, NCHUNK, body, 0)
        plsc.subcore_barrier()
        pltpu.sync_copy(acc_sh.at[pl.ds(s * 640, 640)],
                        deg_out.at[c, k, pl.ds(s * 640, 640)])
        plsc.subcore_barrier()


@functools.cache
def _sc_norm_kernel():
    return pl.kernel(
        _sc_norm_body,
        out_type=jax.ShapeDtypeStruct((K, NW * NCHUNK, C), jnp.float32),
        mesh=_mesh(),
        compiler_params=pltpu.CompilerParams(use_tc_tiling_on_sc=False, needs_layout_passes=False),
        scratch_types=[
            pltpu.VMEM((NPAD,), jnp.float32),
            pltpu.VMEM((NCHUNK, C), jnp.int32),
            pltpu.VMEM((NCHUNK, C), jnp.int32),
            pltpu.VMEM((NCHUNK, C), jnp.float32),
            pltpu.VMEM((NCHUNK, C), jnp.float32),
        ],
    )


def _sc_norm_body(src_hbm, dst_hbm, w_hbm, dis_hbm, norm_out,
                  dis_v, src_v, dst_v, w_v, nrm_v):
    w, c, s = _wid()
    for k in range(K):
        pltpu.sync_copy(dis_hbm.at[k], dis_v)
        pltpu.sync_copy(src_hbm.at[k, pl.ds(w * NCHUNK, NCHUNK)], src_v)
        pltpu.sync_copy(dst_hbm.at[k, pl.ds(w * NCHUNK, NCHUNK)], dst_v)
        pltpu.sync_copy(w_hbm.at[k, pl.ds(w * NCHUNK, NCHUNK)], w_v)

        def body(j, _):
            for t in range(C // 16):
                s16 = src_v[j, pl.ds(t * 16, 16)]
                d16 = dst_v[j, pl.ds(t * 16, 16)]
                w16 = w_v[j, pl.ds(t * 16, 16)]
                a = plsc.load_gather(dis_v, [s16])
                b = plsc.load_gather(dis_v, [d16])
                nrm_v[j, pl.ds(t * 16, 16)] = a * w16 * b
            return 0
        lax.fori_loop(0, NCHUNK, body, 0)
        pltpu.sync_copy(nrm_v, norm_out.at[k, pl.ds(w * NCHUNK, NCHUNK)])


@functools.cache
def _sc_msg_kernel():
    return pl.kernel(
        _sc_msg_body,
        out_type=jax.ShapeDtypeStruct((NC, K, NPAD, D), jnp.float32),
        mesh=_mesh(),
        compiler_params=pltpu.CompilerParams(use_tc_tiling_on_sc=False, needs_layout_passes=False),
        scratch_types=[
            pltpu.VMEM((SUB, C), jnp.int32),
            pltpu.VMEM((SUB, C), jnp.int32),
            pltpu.VMEM((SUB, C), jnp.float32),
            pltpu.VMEM((C,), jnp.int32),
            pltpu.VMEM((C,), jnp.int32),
            pltpu.VMEM((C, D), jnp.float32),
            pltpu.VMEM((ZR, D), jnp.float32),
            pltpu.VMEM_SHARED((NPAD, D), jnp.float32),
            pltpu.SemaphoreType.DMA,
        ],
    )


def _sc_msg_body(xw_hbm, src_hbm, dst_hbm, norm_hbm, out_hbm,
                 src_v, dst_v, nrm_v, sidx_v, didx_v, rows_v, zb_v,
                 acc_sh, sem):
    w, c, s = _wid()
    zero16 = jnp.zeros((16,), jnp.float32)

    def zb_body(i, _):
        for t in range(D // 16):
            zb_v[i, pl.ds(t * 16, 16)] = zero16
        return 0
    lax.fori_loop(0, ZR, zb_body, 0)

    for k in range(K):
        for z in range(640 // ZR):
            pltpu.sync_copy(zb_v, acc_sh.at[pl.ds(s * 640 + z * ZR, ZR)])
        plsc.subcore_barrier()

        for r in range(NCHUNK // SUB):
            base = w * NCHUNK + r * SUB
            pltpu.sync_copy(src_hbm.at[k, pl.ds(base, SUB)], src_v)
            pltpu.sync_copy(dst_hbm.at[k, pl.ds(base, SUB)], dst_v)
            pltpu.sync_copy(norm_hbm.at[k, pl.ds(base, SUB)], nrm_v)

            def body(j, _):
                for t in range(C // 16):
                    sidx_v[pl.ds(t * 16, 16)] = src_v[j, pl.ds(t * 16, 16)]
                    didx_v[pl.ds(t * 16, 16)] = dst_v[j, pl.ds(t * 16, 16)]
                pltpu.async_copy(xw_hbm.at[sidx_v], rows_v, sem).wait()
                j16 = jnp.full((16,), j, jnp.int32)

                def scale(e, _):
                    e16 = jnp.full((16,), e, jnp.int32)
                    nb = plsc.load_gather(nrm_v, [j16, e16])
                    for t in range(D // 16):
                        rows_v[e, pl.ds(t * 16, 16)] = (
                            rows_v[e, pl.ds(t * 16, 16)] * nb)
                    return 0
                lax.fori_loop(0, C, scale, 0)
                pltpu.sync_copy(rows_v, acc_sh.at[didx_v], add=True)
                return 0
            lax.fori_loop(0, SUB, body, 0)
        plsc.subcore_barrier()
        pltpu.sync_copy(acc_sh.at[pl.ds(s * 640, 640)],
                        out_hbm.at[c, k, pl.ds(s * 640, 640)])
        plsc.subcore_barrier()


_BLK = 2000
_HIGH = lax.Precision.HIGHEST


def _bn_mm_body(x_ref, g_ref, b_ref, w_ref, xbn_ref, xw_ref):
    x = x_ref[...]
    mean = jnp.mean(x, axis=0, keepdims=True)
    var = jnp.mean((x - mean) ** 2, axis=0, keepdims=True)
    xbn = (x - mean) * lax.rsqrt(var + 1e-5) * g_ref[...] + b_ref[...]
    xbn_ref[...] = xbn
    xw_ref[...] = jnp.dot(xbn, w_ref[...], precision=_HIGH)


def _tc_bn_mm(x, g, b, w0):
    return pl.pallas_call(
        _bn_mm_body,
        out_shape=(jax.ShapeDtypeStruct((N, D), jnp.float32),
                   jax.ShapeDtypeStruct((N, D), jnp.float32)),
    )(x, g.reshape(1, D), b.reshape(1, D), w0)


def _dis_body(deg_ref, dis_ref):
    deg = deg_ref[0] + deg_ref[1]
    dis_ref[...] = jnp.where(deg > 0, lax.rsqrt(deg), 0.0)


def _tc_dis(deg):
    return pl.pallas_call(
        _dis_body,
        out_shape=jax.ShapeDtypeStruct((K, NPAD // D, D), jnp.float32),
    )(deg.reshape(NC, K, NPAD // D, D))


def _att_body(x_ref, p_ref, b_ref, o_ref):
    x = x_ref[...]
    p = p_ref[...]
    h = p[0] + p[1] + b_ref[...]
    sc = jnp.sum(x[None] * h, axis=-1, keepdims=True)
    m = jnp.max(sc, axis=0, keepdims=True)
    ex = jnp.exp(sc - m)
    a = ex / jnp.sum(ex, axis=0, keepdims=True)
    o_ref[...] = jnp.maximum(jnp.sum(a * h, axis=0), 0.0)


def _att_mm_body(x_ref, p_ref, b_ref, w_ref, o_ref, ow_ref):
    _att_body(x_ref, p_ref, b_ref, o_ref)
    ow_ref[...] = jnp.dot(o_ref[...], w_ref[...], precision=_HIGH)


def _tc_att_mm(x, parts, bias, wnext):
    grid = (N // _BLK,)
    return pl.pallas_call(
        _att_mm_body,
        grid=grid,
        in_specs=[
            pl.BlockSpec((_BLK, D), lambda i: (i, 0)),
            pl.BlockSpec((NC, K, _BLK, D), lambda i: (0, 0, i, 0)),
            pl.BlockSpec((1, D), lambda i: (0, 0)),
            pl.BlockSpec((D, D), lambda i: (0, 0)),
        ],
        out_specs=(pl.BlockSpec((_BLK, D), lambda i: (i, 0)),
                   pl.BlockSpec((_BLK, D), lambda i: (i, 0))),
        out_shape=(jax.ShapeDtypeStruct((N, D), jnp.float32),
                   jax.ShapeDtypeStruct((N, D), jnp.float32)),
    )(x, parts, bias.reshape(1, D), wnext)


def _att_mlp_body(x_ref, p_ref, b_ref, w1_ref, b1_ref, w2_ref, b2_ref,
                  o_ref, scratch_ref):
    _att_body(x_ref, p_ref, b_ref, scratch_ref)
    h = jnp.dot(scratch_ref[...], w1_ref[...], precision=_HIGH) + b1_ref[...]
    h = jnp.where(h >= 0, h, 0.01 * h)
    o = jnp.dot(h, w2_ref[...], precision=_HIGH) + b2_ref[...]
    o_ref[...] = jnp.where(o >= 0, o, 0.01 * o)


def _tc_att_mlp(x, parts, bias, w1, b1, w2, b2):
    grid = (N // _BLK,)
    return pl.pallas_call(
        _att_mlp_body,
        grid=grid,
        in_specs=[
            pl.BlockSpec((_BLK, D), lambda i: (i, 0)),
            pl.BlockSpec((NC, K, _BLK, D), lambda i: (0, 0, i, 0)),
            pl.BlockSpec((1, D), lambda i: (0, 0)),
            pl.BlockSpec((D, H1), lambda i: (0, 0)),
            pl.BlockSpec((1, H1), lambda i: (0, 0)),
            pl.BlockSpec((H1, H2), lambda i: (0, 0)),
            pl.BlockSpec((1, H2), lambda i: (0, 0)),
        ],
        out_specs=pl.BlockSpec((_BLK, H2), lambda i: (i, 0)),
        out_shape=jax.ShapeDtypeStruct((N, H2), jnp.float32),
        scratch_shapes=[pltpu.VMEM((_BLK, D), jnp.float32)],
    )(x, parts, bias.reshape(1, D), w1, b1.reshape(1, H1), w2,
      b2.reshape(1, H2))


def kernel(x, edges_index, edges_weight, bn_g, bn_b, Wg, bg, W1, b1, W2, b2):
    src = edges_index[:, 0].reshape(K, NW * NCHUNK, C).astype(jnp.int32)
    dst = edges_index[:, 1].reshape(K, NW * NCHUNK, C).astype(jnp.int32)
    wre = edges_weight.reshape(K, NW * NCHUNK, C)

    deg = _sc_deg_kernel()(dst, wre)
    dis = _tc_dis(deg).reshape(K, NPAD)
    norm = _sc_norm_kernel()(src, dst, wre, dis)

    xbn, xw = _tc_bn_mm(x, bn_g, bn_b, Wg[0])
    xcur = xbn
    for i in range(N_LAYER):
        parts = _sc_msg_kernel()(xw, src, dst, norm)
        if i + 1 < N_LAYER:
            xcur, xw = _tc_att_mm(xcur, parts, bg[i], Wg[i + 1])
        else:
            out = _tc_att_mlp(xcur, parts, bg[i], W1, b1, W2, b2)
    return out


# speedup vs baseline: 16.0532x; 2.1180x over previous
import functools

import jax
import jax.numpy as jnp
from jax import lax
from jax.experimental import pallas as pl
from jax.experimental.pallas import tpu as pltpu
from jax.experimental.pallas import tpu_sc as plsc

N = 10000
E = 320000
K = 3
D = 128
H1 = 128
H2 = 64
N_LAYER = 3

NC = 2
NS = 16
NW = NC * NS
NPAD = 10240
EPW = E // NW
C = 80
NCHUNK = EPW // C
SUB = 25
ZR = 64

def _wid():
    c = lax.axis_index("c")
    s = lax.axis_index("s")
    return s * NC + c, c, s


def _mesh():
    return plsc.VectorSubcoreMesh(
        core_axis_name="c", subcore_axis_name="s",
        num_cores=NC, num_subcores=NS)


@functools.cache
def _sc_deg_kernel():
    return pl.kernel(
        _sc_deg_body,
        out_type=jax.ShapeDtypeStruct((NC, K, NPAD), jnp.float32),
        mesh=_mesh(),
        compiler_params=pltpu.CompilerParams(use_tc_tiling_on_sc=False, needs_layout_passes=False),
        scratch_types=[
            pltpu.VMEM((NCHUNK, C), jnp.int32),
            pltpu.VMEM((NCHUNK, C), jnp.float32),
            pltpu.VMEM((C,), jnp.int32),
            pltpu.VMEM((C,), jnp.float32),
            pltpu.VMEM((640,), jnp.float32),
            pltpu.VMEM_SHARED((NPAD,), jnp.float32),
        ],
    )


def _sc_deg_body(dst_hbm, w_hbm, deg_out, dst_v, w_v, didx_v, w1_v, zb_v,
                 acc_sh):
    w, c, s = _wid()
    zero16 = jnp.zeros((16,), jnp.float32)

    def zb_body(i, _):
        zb_v[pl.ds(i * 16, 16)] = zero16
        return 0
    lax.fori_loop(0, 640 // 16, zb_body, 0)

    for k in range(K):
        pltpu.sync_copy(zb_v, acc_sh.at[pl.ds(s * 640, 640)])
        plsc.subcore_barrier()
        pltpu.sync_copy(dst_hbm.at[k, pl.ds(w * NCHUNK, NCHUNK)], dst_v)
        pltpu.sync_copy(w_hbm.at[k, pl.ds(w * NCHUNK, NCHUNK)], w_v)

        def body(j, _):
            for t in range(C // 16):
                didx_v[pl.ds(t * 16, 16)] = dst_v[j, pl.ds(t * 16, 16)]
                w1_v[pl.ds(t * 16, 16)] = w_v[j, pl.ds(t * 16, 16)]
            pltpu.sync_copy(w1_v, acc_sh.at[didx_v], add=True)
            return 0
        lax.fori_loop(0, NCHUNK, body, 0)
        plsc.subcore_barrier()
        pltpu.sync_copy(acc_sh.at[pl.ds(s * 640, 640)],
                        deg_out.at[c, k, pl.ds(s * 640, 640)])
        plsc.subcore_barrier()


@functools.cache
def _sc_norm_kernel():
    return pl.kernel(
        _sc_norm_body,
        out_type=jax.ShapeDtypeStruct((K, NW * NCHUNK, C), jnp.float32),
        mesh=_mesh(),
        compiler_params=pltpu.CompilerParams(use_tc_tiling_on_sc=False, needs_layout_passes=False),
        scratch_types=[
            pltpu.VMEM((NPAD,), jnp.float32),
            pltpu.VMEM((NCHUNK, C), jnp.int32),
            pltpu.VMEM((NCHUNK, C), jnp.int32),
            pltpu.VMEM((NCHUNK, C), jnp.float32),
            pltpu.VMEM((NCHUNK, C), jnp.float32),
        ],
    )


def _sc_norm_body(src_hbm, dst_hbm, w_hbm, dis_hbm, norm_out,
                  dis_v, src_v, dst_v, w_v, nrm_v):
    w, c, s = _wid()
    for k in range(K):
        pltpu.sync_copy(dis_hbm.at[k], dis_v)
        pltpu.sync_copy(src_hbm.at[k, pl.ds(w * NCHUNK, NCHUNK)], src_v)
        pltpu.sync_copy(dst_hbm.at[k, pl.ds(w * NCHUNK, NCHUNK)], dst_v)
        pltpu.sync_copy(w_hbm.at[k, pl.ds(w * NCHUNK, NCHUNK)], w_v)

        def body(j, _):
            for t in range(C // 16):
                s16 = src_v[j, pl.ds(t * 16, 16)]
                d16 = dst_v[j, pl.ds(t * 16, 16)]
                w16 = w_v[j, pl.ds(t * 16, 16)]
                a = plsc.load_gather(dis_v, [s16])
                b = plsc.load_gather(dis_v, [d16])
                nrm_v[j, pl.ds(t * 16, 16)] = a * w16 * b
            return 0
        lax.fori_loop(0, NCHUNK, body, 0)
        pltpu.sync_copy(nrm_v, norm_out.at[k, pl.ds(w * NCHUNK, NCHUNK)])


@functools.cache
def _sc_msg_kernel():
    return pl.kernel(
        _sc_msg_body,
        out_type=jax.ShapeDtypeStruct((NC, K, NPAD, D), jnp.float32),
        mesh=_mesh(),
        compiler_params=pltpu.CompilerParams(use_tc_tiling_on_sc=False, needs_layout_passes=False),
        scratch_types=[
            pltpu.VMEM((SUB, C), jnp.int32),
            pltpu.VMEM((SUB, C), jnp.int32),
            pltpu.VMEM((SUB, C), jnp.float32),
            pltpu.VMEM((C,), jnp.int32),
            pltpu.VMEM((C,), jnp.int32),
            pltpu.VMEM((C,), jnp.int32),
            pltpu.VMEM((C,), jnp.int32),
            pltpu.VMEM((C,), jnp.int32),
            pltpu.VMEM((C,), jnp.int32),
            pltpu.VMEM((C, D), jnp.float32),
            pltpu.VMEM((C, D), jnp.float32),
            pltpu.VMEM((C, D), jnp.float32),
            pltpu.VMEM((ZR, D), jnp.float32),
            pltpu.VMEM_SHARED((NPAD, D), jnp.float32),
            pltpu.SemaphoreType.DMA,
            pltpu.SemaphoreType.DMA,
            pltpu.SemaphoreType.DMA,
            pltpu.SemaphoreType.DMA,
            pltpu.SemaphoreType.DMA,
            pltpu.SemaphoreType.DMA,
        ],
    )


def _sc_msg_body(xw_hbm, src_hbm, dst_hbm, norm_hbm, out_hbm,
                 src_v, dst_v, nrm_v, sidx0, sidx1, sidx2,
                 didx0, didx1, didx2, rows0, rows1, rows2, zb_v, acc_sh,
                 sg0, sg1, sg2, ss0, ss1, ss2):
    w, c, s = _wid()
    zero16 = jnp.zeros((16,), jnp.float32)
    sidx = (sidx0, sidx1, sidx2)
    didx = (didx0, didx1, didx2)
    rows = (rows0, rows1, rows2)
    sg = (sg0, sg1, sg2)
    ss = (ss0, ss1, ss2)

    def zb_body(i, _):
        for t in range(D // 16):
            zb_v[i, pl.ds(t * 16, 16)] = zero16
        return 0
    lax.fori_loop(0, ZR, zb_body, 0)

    def stage(j, b):
        for t in range(C // 16):
            sidx[b][pl.ds(t * 16, 16)] = src_v[j, pl.ds(t * 16, 16)]
            didx[b][pl.ds(t * 16, 16)] = dst_v[j, pl.ds(t * 16, 16)]

    def fire_gather(b):
        return pltpu.async_copy(xw_hbm.at[sidx[b]], rows[b], sg[b])

    def wait_gather(b):
        pltpu.make_async_copy(xw_hbm.at[sidx[b]], rows[b], sg[b]).wait()

    def fire_scatter(b):
        pltpu.async_copy(rows[b], acc_sh.at[didx[b]], ss[b], add=True)

    def drain_scatter(b):
        pltpu.make_async_copy(rows[b], acc_sh.at[didx[b]], ss[b]).wait()

    def scale_chunk(j, b):
        j16 = jnp.full((16,), j, jnp.int32)
        rb = rows[b]

        @plsc.parallel_loop(0, C, unroll=4)
        def _(e):
            e16 = jnp.full((16,), e, jnp.int32)
            nb = plsc.load_gather(nrm_v, [j16, e16])
            for t in range(D // 16):
                rb[e, pl.ds(t * 16, 16)] = rb[e, pl.ds(t * 16, 16)] * nb

    def slot(j, b, first, may_fire_ahead):
        wait_gather(b)
        scale_chunk(j, b)
        fire_scatter(b)
        if not first:
            drain_scatter((b + 2) % 3)
        if may_fire_ahead:
            @pl.when(j + 2 <= SUB - 1)
            def _():
                stage(j + 2, (b + 2) % 3)
                fire_gather((b + 2) % 3)

    for k in range(K):
        for z in range(640 // ZR):
            pltpu.sync_copy(zb_v, acc_sh.at[pl.ds(s * 640 + z * ZR, ZR)])
        plsc.subcore_barrier()

        def rbody(r, _):
            base = w * NCHUNK + r * SUB
            pltpu.sync_copy(src_hbm.at[k, pl.ds(base, SUB)], src_v)
            pltpu.sync_copy(dst_hbm.at[k, pl.ds(base, SUB)], dst_v)
            pltpu.sync_copy(norm_hbm.at[k, pl.ds(base, SUB)], nrm_v)

            stage(0, 0)
            fire_gather(0)
            stage(1, 1)
            fire_gather(1)
            slot(0, 0, True, True)

            def body(g, _):
                j = 3 * g + 1
                slot(j, 1, False, True)
                slot(j + 1, 2, False, True)
                slot(j + 2, 0, False, True)
                return 0
            lax.fori_loop(0, (SUB - 1) // 3, body, 0)
            drain_scatter(0)
            return 0
        lax.fori_loop(0, NCHUNK // SUB, rbody, 0)
        plsc.subcore_barrier()
        pltpu.sync_copy(acc_sh.at[pl.ds(s * 640, 640)],
                        out_hbm.at[c, k, pl.ds(s * 640, 640)])
        plsc.subcore_barrier()


_BLK = 2000
_HIGH = lax.Precision.HIGHEST


def _bn_mm_body(x_ref, g_ref, b_ref, w_ref, xbn_ref, xw_ref):
    x = x_ref[...]
    mean = jnp.mean(x, axis=0, keepdims=True)
    var = jnp.mean((x - mean) ** 2, axis=0, keepdims=True)
    xbn = (x - mean) * lax.rsqrt(var + 1e-5) * g_ref[...] + b_ref[...]
    xbn_ref[...] = xbn
    xw_ref[...] = jnp.dot(xbn, w_ref[...], precision=_HIGH)


def _tc_bn_mm(x, g, b, w0):
    return pl.pallas_call(
        _bn_mm_body,
        out_shape=(jax.ShapeDtypeStruct((N, D), jnp.float32),
                   jax.ShapeDtypeStruct((N, D), jnp.float32)),
    )(x, g.reshape(1, D), b.reshape(1, D), w0)


def _dis_body(deg_ref, dis_ref):
    deg = deg_ref[0] + deg_ref[1]
    dis_ref[...] = jnp.where(deg > 0, lax.rsqrt(deg), 0.0)


def _tc_dis(deg):
    return pl.pallas_call(
        _dis_body,
        out_shape=jax.ShapeDtypeStruct((K, NPAD // D, D), jnp.float32),
    )(deg.reshape(NC, K, NPAD // D, D))


def _att_body(x_ref, p_ref, b_ref, o_ref):
    x = x_ref[...]
    p = p_ref[...]
    h = p[0] + p[1] + b_ref[...]
    sc = jnp.sum(x[None] * h, axis=-1, keepdims=True)
    m = jnp.max(sc, axis=0, keepdims=True)
    ex = jnp.exp(sc - m)
    a = ex / jnp.sum(ex, axis=0, keepdims=True)
    o_ref[...] = jnp.maximum(jnp.sum(a * h, axis=0), 0.0)


def _att_mm_body(x_ref, p_ref, b_ref, w_ref, o_ref, ow_ref):
    _att_body(x_ref, p_ref, b_ref, o_ref)
    ow_ref[...] = jnp.dot(o_ref[...], w_ref[...], precision=_HIGH)


def _tc_att_mm(x, parts, bias, wnext):
    grid = (N // _BLK,)
    return pl.pallas_call(
        _att_mm_body,
        grid=grid,
        in_specs=[
            pl.BlockSpec((_BLK, D), lambda i: (i, 0)),
            pl.BlockSpec((NC, K, _BLK, D), lambda i: (0, 0, i, 0)),
            pl.BlockSpec((1, D), lambda i: (0, 0)),
            pl.BlockSpec((D, D), lambda i: (0, 0)),
        ],
        out_specs=(pl.BlockSpec((_BLK, D), lambda i: (i, 0)),
                   pl.BlockSpec((_BLK, D), lambda i: (i, 0))),
        out_shape=(jax.ShapeDtypeStruct((N, D), jnp.float32),
                   jax.ShapeDtypeStruct((N, D), jnp.float32)),
    )(x, parts, bias.reshape(1, D), wnext)


def _att_mlp_body(x_ref, p_ref, b_ref, w1_ref, b1_ref, w2_ref, b2_ref,
                  o_ref, scratch_ref):
    _att_body(x_ref, p_ref, b_ref, scratch_ref)
    h = jnp.dot(scratch_ref[...], w1_ref[...], precision=_HIGH) + b1_ref[...]
    h = jnp.where(h >= 0, h, 0.01 * h)
    o = jnp.dot(h, w2_ref[...], precision=_HIGH) + b2_ref[...]
    o_ref[...] = jnp.where(o >= 0, o, 0.01 * o)


def _tc_att_mlp(x, parts, bias, w1, b1, w2, b2):
    grid = (N // _BLK,)
    return pl.pallas_call(
        _att_mlp_body,
        grid=grid,
        in_specs=[
            pl.BlockSpec((_BLK, D), lambda i: (i, 0)),
            pl.BlockSpec((NC, K, _BLK, D), lambda i: (0, 0, i, 0)),
            pl.BlockSpec((1, D), lambda i: (0, 0)),
            pl.BlockSpec((D, H1), lambda i: (0, 0)),
            pl.BlockSpec((1, H1), lambda i: (0, 0)),
            pl.BlockSpec((H1, H2), lambda i: (0, 0)),
            pl.BlockSpec((1, H2), lambda i: (0, 0)),
        ],
        out_specs=pl.BlockSpec((_BLK, H2), lambda i: (i, 0)),
        out_shape=jax.ShapeDtypeStruct((N, H2), jnp.float32),
        scratch_shapes=[pltpu.VMEM((_BLK, D), jnp.float32)],
    )(x, parts, bias.reshape(1, D), w1, b1.reshape(1, H1), w2,
      b2.reshape(1, H2))


def kernel(x, edges_index, edges_weight, bn_g, bn_b, Wg, bg, W1, b1, W2, b2):
    src = edges_index[:, 0].reshape(K, NW * NCHUNK, C).astype(jnp.int32)
    dst = edges_index[:, 1].reshape(K, NW * NCHUNK, C).astype(jnp.int32)
    wre = edges_weight.reshape(K, NW * NCHUNK, C)

    deg = _sc_deg_kernel()(dst, wre)
    dis = _tc_dis(deg).reshape(K, NPAD)
    norm = _sc_norm_kernel()(src, dst, wre, dis)

    xbn, xw = _tc_bn_mm(x, bn_g, bn_b, Wg[0])
    xcur = xbn
    for i in range(N_LAYER):
        parts = _sc_msg_kernel()(xw, src, dst, norm)
        if i + 1 < N_LAYER:
            xcur, xw = _tc_att_mm(xcur, parts, bg[i], Wg[i + 1])
        else:
            out = _tc_att_mlp(xcur, parts, bg[i], W1, b1, W2, b2)
    return out


# speedup vs baseline: 17.3604x; 1.0814x over previous
import functools

import jax
import jax.numpy as jnp
from jax import lax
from jax.experimental import pallas as pl
from jax.experimental.pallas import tpu as pltpu
from jax.experimental.pallas import tpu_sc as plsc

N = 10000
E = 320000
K = 3
D = 128
H1 = 128
H2 = 64
N_LAYER = 3

NC = 2
NS = 16
NW = NC * NS
NPAD = 10240
EPW = E // NW
C = 80
NCHUNK = EPW // C
SUB = 25
ZR = 64

def _wid():
    c = lax.axis_index("c")
    s = lax.axis_index("s")
    return s * NC + c, c, s


def _mesh():
    return plsc.VectorSubcoreMesh(
        core_axis_name="c", subcore_axis_name="s",
        num_cores=NC, num_subcores=NS)


@functools.cache
def _sc_deg_kernel():
    return pl.kernel(
        _sc_deg_body,
        out_type=jax.ShapeDtypeStruct((NC, K, NPAD), jnp.float32),
        mesh=_mesh(),
        compiler_params=pltpu.CompilerParams(use_tc_tiling_on_sc=False, needs_layout_passes=False),
        scratch_types=[
            pltpu.VMEM((NCHUNK, C), jnp.int32),
            pltpu.VMEM((NCHUNK, C), jnp.float32),
            pltpu.VMEM((C,), jnp.int32),
            pltpu.VMEM((C,), jnp.float32),
            pltpu.VMEM((640,), jnp.float32),
            pltpu.VMEM_SHARED((NPAD,), jnp.float32),
        ],
    )


def _sc_deg_body(dst_hbm, w_hbm, deg_out, dst_v, w_v, didx_v, w1_v, zb_v,
                 acc_sh):
    w, c, s = _wid()
    zero16 = jnp.zeros((16,), jnp.float32)

    def zb_body(i, _):
        zb_v[pl.ds(i * 16, 16)] = zero16
        return 0
    lax.fori_loop(0, 640 // 16, zb_body, 0)

    for k in range(K):
        pltpu.sync_copy(zb_v, acc_sh.at[pl.ds(s * 640, 640)])
        plsc.subcore_barrier()
        pltpu.sync_copy(dst_hbm.at[k, pl.ds(w * NCHUNK, NCHUNK)], dst_v)
        pltpu.sync_copy(w_hbm.at[k, pl.ds(w * NCHUNK, NCHUNK)], w_v)

        def body(j, _):
            for t in range(C // 16):
                didx_v[pl.ds(t * 16, 16)] = dst_v[j, pl.ds(t * 16, 16)]
                w1_v[pl.ds(t * 16, 16)] = w_v[j, pl.ds(t * 16, 16)]
            pltpu.sync_copy(w1_v, acc_sh.at[didx_v], add=True)
            return 0
        lax.fori_loop(0, NCHUNK, body, 0)
        plsc.subcore_barrier()
        pltpu.sync_copy(acc_sh.at[pl.ds(s * 640, 640)],
                        deg_out.at[c, k, pl.ds(s * 640, 640)])
        plsc.subcore_barrier()


@functools.cache
def _sc_norm_kernel():
    return pl.kernel(
        _sc_norm_body,
        out_type=jax.ShapeDtypeStruct((K, NW * NCHUNK, C), jnp.float32),
        mesh=_mesh(),
        compiler_params=pltpu.CompilerParams(use_tc_tiling_on_sc=False, needs_layout_passes=False),
        scratch_types=[
            pltpu.VMEM((NPAD,), jnp.float32),
            pltpu.VMEM((NCHUNK, C), jnp.int32),
            pltpu.VMEM((NCHUNK, C), jnp.int32),
            pltpu.VMEM((NCHUNK, C), jnp.float32),
            pltpu.VMEM((NCHUNK, C), jnp.float32),
        ],
    )


def _sc_norm_body(src_hbm, dst_hbm, w_hbm, dis_hbm, norm_out,
                  dis_v, src_v, dst_v, w_v, nrm_v):
    w, c, s = _wid()
    for k in range(K):
        pltpu.sync_copy(dis_hbm.at[k], dis_v)
        pltpu.sync_copy(src_hbm.at[k, pl.ds(w * NCHUNK, NCHUNK)], src_v)
        pltpu.sync_copy(dst_hbm.at[k, pl.ds(w * NCHUNK, NCHUNK)], dst_v)
        pltpu.sync_copy(w_hbm.at[k, pl.ds(w * NCHUNK, NCHUNK)], w_v)

        def body(j, _):
            for t in range(C // 16):
                s16 = src_v[j, pl.ds(t * 16, 16)]
                d16 = dst_v[j, pl.ds(t * 16, 16)]
                w16 = w_v[j, pl.ds(t * 16, 16)]
                a = plsc.load_gather(dis_v, [s16])
                b = plsc.load_gather(dis_v, [d16])
                nrm_v[j, pl.ds(t * 16, 16)] = a * w16 * b
            return 0
        lax.fori_loop(0, NCHUNK, body, 0)
        pltpu.sync_copy(nrm_v, norm_out.at[k, pl.ds(w * NCHUNK, NCHUNK)])


GBUF = 3
FBUF = 2


@functools.cache
def _sc_msg_kernel():
    return pl.kernel(
        _sc_msg_body,
        out_type=jax.ShapeDtypeStruct((NC, K, NPAD, D), jnp.float32),
        mesh=_mesh(),
        compiler_params=pltpu.CompilerParams(use_tc_tiling_on_sc=False, needs_layout_passes=False),
        scratch_types=(
            [pltpu.VMEM((SUB, C), jnp.int32),
             pltpu.VMEM((SUB, C), jnp.int32),
             pltpu.VMEM((SUB, C), jnp.float32)]
            + [pltpu.VMEM((C,), jnp.int32) for _ in range(GBUF)]
            + [pltpu.VMEM((C,), jnp.int32) for _ in range(FBUF)]
            + [pltpu.VMEM((C, D // 2), jnp.int32) for _ in range(GBUF)]
            + [pltpu.VMEM((C, D), jnp.float32) for _ in range(FBUF)]
            + [pltpu.VMEM_SHARED((NPAD, D), jnp.float32)]
            + [pltpu.SemaphoreType.DMA for _ in range(GBUF + FBUF)]
        ),
    )


def _sc_msg_body(xw_hbm, src_hbm, dst_hbm, norm_hbm, out_hbm, *refs):
    src_v, dst_v, nrm_v = refs[0:3]
    sidx = refs[3:3 + GBUF]
    didx = refs[3 + GBUF:3 + GBUF + FBUF]
    rows_bf = refs[3 + GBUF + FBUF:3 + 2 * GBUF + FBUF]
    rows = refs[3 + 2 * GBUF + FBUF:3 + 2 * GBUF + 2 * FBUF]
    acc_sh = refs[3 + 2 * GBUF + 2 * FBUF]
    sems = refs[4 + 2 * GBUF + 2 * FBUF:]
    sg = sems[0:GBUF]
    ss = sems[GBUF:GBUF + FBUF]
    w, c, s = _wid()
    zero16 = jnp.zeros((16,), jnp.float32)

    def stage_src(j, b):
        for t in range(C // 16):
            sidx[b][pl.ds(t * 16, 16)] = src_v[j, pl.ds(t * 16, 16)]

    def stage_dst(j, f):
        for t in range(C // 16):
            didx[f][pl.ds(t * 16, 16)] = dst_v[j, pl.ds(t * 16, 16)]

    def fire_gather(b):
        return pltpu.async_copy(xw_hbm.at[sidx[b]], rows_bf[b], sg[b])

    def wait_gather(b):
        pltpu.make_async_copy(xw_hbm.at[sidx[b]], rows_bf[b], sg[b]).wait()

    def fire_scatter(f):
        pltpu.async_copy(rows[f], acc_sh.at[didx[f]], ss[f], add=True)

    def drain_scatter(f):
        pltpu.make_async_copy(rows[f], acc_sh.at[didx[f]], ss[f]).wait()

    def scale_chunk(j, b, f):
        j16 = jnp.full((16,), j, jnp.int32)
        rbi = rows_bf[b]
        rbo = rows[f]

        @plsc.parallel_loop(0, C, unroll=4)
        def _(e):
            e16 = jnp.full((16,), e, jnp.int32)
            nb = plsc.load_gather(nrm_v, [j16, e16])
            for t in range(D // 32):
                vb = plsc.bitcast(rbi[e, pl.ds(t * 16, 16)], jnp.bfloat16)
                av, bv = plsc.unpack(vb, format=plsc.PackFormat.INTERLEAVED,
                                     preferred_element_type=jnp.float32)
                rbo[e, pl.ds(t * 32, 16)] = av * nb
                rbo[e, pl.ds(t * 32 + 16, 16)] = bv * nb

    def slot(j, b, f, drain, fire):
        wait_gather(b)
        if drain:
            drain_scatter(f)
        stage_dst(j, f)
        scale_chunk(j, b, f)
        fire_scatter(f)
        if fire:
            stage_src(j + 2, (b + 2) % GBUF)
            fire_gather((b + 2) % GBUF)

    for k in range(K):
        def zb_body(i, _):
            for t in range(D // 16):
                rows[0][i, pl.ds(t * 16, 16)] = zero16
            return 0
        lax.fori_loop(0, C, zb_body, 0)
        for z in range(640 // C):
            pltpu.sync_copy(rows[0], acc_sh.at[pl.ds(s * 640 + z * C, C)])
        plsc.subcore_barrier()

        def rbody(r, _):
            base = w * NCHUNK + r * SUB
            pltpu.sync_copy(src_hbm.at[k, pl.ds(base, SUB)], src_v)
            pltpu.sync_copy(dst_hbm.at[k, pl.ds(base, SUB)], dst_v)
            pltpu.sync_copy(norm_hbm.at[k, pl.ds(base, SUB)], nrm_v)

            stage_src(0, 0)
            fire_gather(0)
            stage_src(1, 1)
            fire_gather(1)
            slot(0, 0, 0, False, True)
            slot(1, 1, 1, False, True)

            def body(g, _):
                j = 6 * g + 2
                for u in range(6):
                    slot(j + u, (2 + u) % GBUF, u % FBUF, True, True)
                return 0
            lax.fori_loop(0, 3, body, 0)
            slot(20, 2, 0, True, True)
            slot(21, 0, 1, True, True)
            slot(22, 1, 0, True, True)
            slot(23, 2, 1, True, False)
            slot(24, 0, 0, True, False)
            drain_scatter(1)
            drain_scatter(0)
            return 0
        lax.fori_loop(0, NCHUNK // SUB, rbody, 0)
        plsc.subcore_barrier()
        pltpu.sync_copy(acc_sh.at[pl.ds(s * 640, 640)],
                        out_hbm.at[c, k, pl.ds(s * 640, 640)])
        plsc.subcore_barrier()


_BLK = 2000
_HIGH = lax.Precision.HIGHEST


def _bn_mm_body(x_ref, g_ref, b_ref, w_ref, xbn_ref, xw_ref):
    x = x_ref[...]
    mean = jnp.mean(x, axis=0, keepdims=True)
    var = jnp.mean((x - mean) ** 2, axis=0, keepdims=True)
    xbn = (x - mean) * lax.rsqrt(var + 1e-5) * g_ref[...] + b_ref[...]
    xbn_ref[...] = xbn
    xw_ref[...] = jnp.dot(xbn, w_ref[...], precision=_HIGH)


def _tc_bn_mm(x, g, b, w0):
    return pl.pallas_call(
        _bn_mm_body,
        out_shape=(jax.ShapeDtypeStruct((N, D), jnp.float32),
                   jax.ShapeDtypeStruct((N, D), jnp.float32)),
    )(x, g.reshape(1, D), b.reshape(1, D), w0)


def _dis_body(deg_ref, dis_ref):
    deg = deg_ref[0] + deg_ref[1]
    dis_ref[...] = jnp.where(deg > 0, lax.rsqrt(deg), 0.0)


def _tc_dis(deg):
    return pl.pallas_call(
        _dis_body,
        out_shape=jax.ShapeDtypeStruct((K, NPAD // D, D), jnp.float32),
    )(deg.reshape(NC, K, NPAD // D, D))


def _att_body(x_ref, p_ref, b_ref, o_ref):
    x = x_ref[...]
    p = p_ref[...]
    h = p[0] + p[1] + b_ref[...]
    sc = jnp.sum(x[None] * h, axis=-1, keepdims=True)
    m = jnp.max(sc, axis=0, keepdims=True)
    ex = jnp.exp(sc - m)
    a = ex / jnp.sum(ex, axis=0, keepdims=True)
    o_ref[...] = jnp.maximum(jnp.sum(a * h, axis=0), 0.0)


def _att_mm_body(x_ref, p_ref, b_ref, w_ref, o_ref, ow_ref):
    _att_body(x_ref, p_ref, b_ref, o_ref)
    ow_ref[...] = jnp.dot(o_ref[...], w_ref[...], precision=_HIGH)


def _tc_att_mm(x, parts, bias, wnext):
    grid = (N // _BLK,)
    return pl.pallas_call(
        _att_mm_body,
        grid=grid,
        in_specs=[
            pl.BlockSpec((_BLK, D), lambda i: (i, 0)),
            pl.BlockSpec((NC, K, _BLK, D), lambda i: (0, 0, i, 0)),
            pl.BlockSpec((1, D), lambda i: (0, 0)),
            pl.BlockSpec((D, D), lambda i: (0, 0)),
        ],
        out_specs=(pl.BlockSpec((_BLK, D), lambda i: (i, 0)),
                   pl.BlockSpec((_BLK, D), lambda i: (i, 0))),
        out_shape=(jax.ShapeDtypeStruct((N, D), jnp.float32),
                   jax.ShapeDtypeStruct((N, D), jnp.float32)),
    )(x, parts, bias.reshape(1, D), wnext)


def _att_mlp_body(x_ref, p_ref, b_ref, w1_ref, b1_ref, w2_ref, b2_ref,
                  o_ref, scratch_ref):
    _att_body(x_ref, p_ref, b_ref, scratch_ref)
    h = jnp.dot(scratch_ref[...], w1_ref[...], precision=_HIGH) + b1_ref[...]
    h = jnp.where(h >= 0, h, 0.01 * h)
    o = jnp.dot(h, w2_ref[...], precision=_HIGH) + b2_ref[...]
    o_ref[...] = jnp.where(o >= 0, o, 0.01 * o)


def _tc_att_mlp(x, parts, bias, w1, b1, w2, b2):
    grid = (N // _BLK,)
    return pl.pallas_call(
        _att_mlp_body,
        grid=grid,
        in_specs=[
            pl.BlockSpec((_BLK, D), lambda i: (i, 0)),
            pl.BlockSpec((NC, K, _BLK, D), lambda i: (0, 0, i, 0)),
            pl.BlockSpec((1, D), lambda i: (0, 0)),
            pl.BlockSpec((D, H1), lambda i: (0, 0)),
            pl.BlockSpec((1, H1), lambda i: (0, 0)),
            pl.BlockSpec((H1, H2), lambda i: (0, 0)),
            pl.BlockSpec((1, H2), lambda i: (0, 0)),
        ],
        out_specs=pl.BlockSpec((_BLK, H2), lambda i: (i, 0)),
        out_shape=jax.ShapeDtypeStruct((N, H2), jnp.float32),
        scratch_shapes=[pltpu.VMEM((_BLK, D), jnp.float32)],
    )(x, parts, bias.reshape(1, D), w1, b1.reshape(1, H1), w2,
      b2.reshape(1, H2))


_PERM = [0] * D
for _t in range(D // 32):
    for _i in range(16):
        _PERM[32 * _t + 2 * _i] = 32 * _t + _i
        _PERM[32 * _t + 2 * _i + 1] = 32 * _t + 16 + _i


def _pack_rows(xw):
    xwb = xw[:, jnp.array(_PERM, jnp.int32)].astype(jnp.bfloat16)
    return lax.bitcast_convert_type(xwb.reshape(N, D // 2, 2), jnp.int32)


def kernel(x, edges_index, edges_weight, bn_g, bn_b, Wg, bg, W1, b1, W2, b2):
    src = edges_index[:, 0].reshape(K, NW * NCHUNK, C).astype(jnp.int32)
    dst = edges_index[:, 1].reshape(K, NW * NCHUNK, C).astype(jnp.int32)
    wre = edges_weight.reshape(K, NW * NCHUNK, C)

    deg = _sc_deg_kernel()(dst, wre)
    dis = _tc_dis(deg).reshape(K, NPAD)
    norm = _sc_norm_kernel()(src, dst, wre, dis)

    xbn, xw = _tc_bn_mm(x, bn_g, bn_b, Wg[0])
    xcur = xbn
    for i in range(N_LAYER):
        parts = _sc_msg_kernel()(_pack_rows(xw), src, dst, norm)
        if i + 1 < N_LAYER:
            xcur, xw = _tc_att_mm(xcur, parts, bg[i], Wg[i + 1])
        else:
            out = _tc_att_mlp(xcur, parts, bg[i], W1, b1, W2, b2)
    return out
